# baseline (speedup 1.0000x reference)
import functools

import jax
import jax.numpy as jnp
from jax.experimental import pallas as pl
from jax.experimental.pallas import tpu as pltpu

NUM_SEG = 128
STAT_W = 136


def _pick_block(n):
    for b in (12800, 6400, 2560, 1280, 640, 320, 160, 80, 40, 16, 8):
        if n % b == 0:
            return b
    return n


def _pick_block_wide(n):
    for b in (32000, 16000, 12800, 6400, 2560, 1280, 640, 320, 160, 80, 40,
              16, 8):
        if n % b == 0:
            return b
    return n


def _lrelu(a):
    return jnp.where(a >= 0, a, 0.01 * a)


def _mask_t(ir_ref, blk):
    return (jax.lax.broadcasted_iota(jnp.int32, (NUM_SEG, blk), 0)
            == ir_ref[...]).astype(jnp.bfloat16)


def _dot(a, b):
    return jax.lax.dot_general(a, b, (((1,), (0,)), ((), ())),
                               preferred_element_type=jnp.float32)


def _split_w(w):
    hi = w.astype(jnp.bfloat16)
    lo = (w - hi.astype(jnp.float32)).astype(jnp.bfloat16)
    return jnp.concatenate([hi, lo], axis=0)


def _dot_split(zb, whl):
    zz = jnp.concatenate([zb, zb], axis=1)
    return _dot(zz, whl)


def _seg_stats(mt, hb, blk):
    hh = jnp.concatenate(
        [hb, hb * hb, jnp.ones((blk, 8), jnp.bfloat16)], axis=1)
    return _dot(mt, hh)


def _accum_and_finalize(s_ref, st, t_ref, nb):
    g = pl.program_id(0)

    @pl.when(g == 0)
    def _():
        s_ref[...] = st

    @pl.when(g > 0)
    def _():
        s_ref[...] += st

    @pl.when(g == nb - 1)
    def _():
        sums = s_ref[:, 0:64]
        sqs = s_ref[:, 64:128]
        cnt = s_ref[:, 128:129]
        mean = jnp.where(cnt > 0, sums / jnp.maximum(cnt, 1.0), 0.0)
        var = jnp.where(
            cnt > 1, (sqs - cnt * mean * mean) / jnp.maximum(cnt - 1.0, 1.0),
            0.0)
        std = jnp.sqrt(jnp.maximum(var, 0.0))
        rstd = 1.0 / (std + 1e-8)
        rhi = rstd.astype(jnp.bfloat16)
        rlo = (rstd - rhi.astype(jnp.float32)).astype(jnp.bfloat16)
        t_ref[...] = jnp.concatenate(
            [rhi, rlo, (mean * rstd).astype(jnp.bfloat16)], axis=1)


def _normalize(hb, mt, t_ref):
    r = jax.lax.dot_general(mt, t_ref[...], (((0,), (0,)), ((), ())),
                            preferred_element_type=jnp.float32)
    rsd = r[:, 0:64] + r[:, 64:128]
    return hb.astype(jnp.float32) * rsd - r[:, 128:192]


def _first_kernel(x_ref, ir_ref, w_ref, b_ref, h_ref, t_ref, s_ref, *,
                  blk, nb):
    xb = x_ref[...].astype(jnp.bfloat16)
    hb = _lrelu(_dot_split(xb, w_ref[...]) + b_ref[...]).astype(jnp.bfloat16)
    h_ref[...] = hb
    _accum_and_finalize(s_ref, _seg_stats(_mask_t(ir_ref, blk), hb, blk),
                        t_ref, nb)


def _mid_kernel(h_ref, ir_ref, t_ref, w_ref, b_ref, ho_ref, to_ref,
                s_ref, *, blk, nb):
    mt = _mask_t(ir_ref, blk)
    zb = _normalize(h_ref[...], mt, t_ref).astype(jnp.bfloat16)
    hb = _lrelu(_dot_split(zb, w_ref[...]) + b_ref[...]).astype(jnp.bfloat16)
    ho_ref[...] = hb
    _accum_and_finalize(s_ref, _seg_stats(mt, hb, blk), to_ref, nb)


def _last_kernel(h_ref, ir_ref, t_ref, w_ref, b_ref, o_ref, *, blk):
    zb = _normalize(h_ref[...], _mask_t(ir_ref, blk), t_ref).astype(jnp.bfloat16)
    o_ref[...] = _dot_split(zb, w_ref[...]) + b_ref[...]


def kernel(x, i, W1, b1, W2, b2, W3, b3, W4, b4):
    n, d = x.shape
    blk1 = _pick_block(n)
    blk2 = _pick_block_wide(n)
    nb1 = n // blk1
    nb2 = n // blk2
    i_row = i.reshape(1, n)

    row_spec = lambda b, w: pl.BlockSpec((b, w), lambda g: (g, 0))
    ir_spec = lambda b: pl.BlockSpec((1, b), lambda g: (0, g))
    full = lambda *s: pl.BlockSpec(s, lambda g: (0,) * len(s))
    tab_shape = jax.ShapeDtypeStruct((NUM_SEG, 192), jnp.bfloat16)
    h_shape = jax.ShapeDtypeStruct((n, 64), jnp.bfloat16)
    scratch = [pltpu.VMEM((NUM_SEG, STAT_W), jnp.float32)]

    h1, t1 = pl.pallas_call(
        functools.partial(_first_kernel, blk=blk1, nb=nb1),
        grid=(nb1,),
        in_specs=[row_spec(blk1, d), ir_spec(blk1), full(2 * d, 64), full(1, 64)],
        out_specs=[row_spec(blk1, 64), full(NUM_SEG, 192)],
        out_shape=[h_shape, tab_shape],
        scratch_shapes=scratch,
    )(x, i_row, _split_w(W1.T), b1.reshape(1, 64))

    mid = pl.pallas_call(
        functools.partial(_mid_kernel, blk=blk2, nb=nb2),
        grid=(nb2,),
        in_specs=[row_spec(blk2, 64), ir_spec(blk2), full(NUM_SEG, 192),
                  full(128, 64), full(1, 64)],
        out_specs=[row_spec(blk2, 64), full(NUM_SEG, 192)],
        out_shape=[h_shape, tab_shape],
        scratch_shapes=scratch,
    )
    h2, t2 = mid(h1, i_row, t1, _split_w(W2.T), b2.reshape(1, 64))
    h3, t3 = mid(h2, i_row, t2, _split_w(W3.T), b3.reshape(1, 64))

    out = pl.pallas_call(
        functools.partial(_last_kernel, blk=blk1),
        grid=(nb1,),
        in_specs=[row_spec(blk1, 64), ir_spec(blk1), full(NUM_SEG, 192),
                  full(128, 3), full(1, 3)],
        out_specs=row_spec(blk1, 3),
        out_shape=jax.ShapeDtypeStruct((n, 3), jnp.float32),
    )(h3, i_row, t3, _split_w(W4.T), b4.reshape(1, 3))
    return out


# speedup vs baseline: 1.0467x; 1.0467x over previous
import functools

import jax
import jax.numpy as jnp
from jax.experimental import pallas as pl
from jax.experimental.pallas import tpu as pltpu

NUM_SEG = 128
TAB_W = 200


def _pick_block(n):
    for b in (12800, 6400, 2560, 1280, 640, 320, 160, 80, 40, 16, 8):
        if n % b == 0:
            return b
    return n


def _pick_block_wide(n):
    for b in (32000, 16000, 12800, 6400, 2560, 1280, 640, 320, 160, 80, 40,
              16, 8):
        if n % b == 0:
            return b
    return n


def _lrelu(a):
    return jnp.maximum(a, 0.01 * a)


def _mask_t(ir_ref, blk):
    return (jax.lax.broadcasted_iota(jnp.int32, (NUM_SEG, blk), 0)
            == ir_ref[...]).astype(jnp.bfloat16)


def _dot(a, b):
    return jax.lax.dot_general(a, b, (((1,), (0,)), ((), ())),
                               preferred_element_type=jnp.float32)


def _dot_t(a, b):
    return jax.lax.dot_general(a, b, (((0,), (0,)), ((), ())),
                               preferred_element_type=jnp.float32)


def _finalize_table(s_ref, cnt, t_ref):
    sums = s_ref[:, 0:64]
    sqs = s_ref[:, 64:128]
    mean = jnp.where(cnt > 0, sums / jnp.maximum(cnt, 1.0), 0.0)
    var = jnp.where(cnt > 1,
                    (sqs - cnt * mean * mean) / jnp.maximum(cnt - 1.0, 1.0),
                    0.0)
    std = jnp.sqrt(jnp.maximum(var, 0.0))
    rstd = 1.0 / (std + 1e-8)
    rhi = rstd.astype(jnp.bfloat16)
    rlo = (rstd - rhi.astype(jnp.float32)).astype(jnp.bfloat16)
    chi = cnt.astype(jnp.bfloat16)
    clo = (cnt - chi.astype(jnp.float32)).astype(jnp.bfloat16)
    t_ref[...] = jnp.concatenate(
        [rhi, rlo, (mean * rstd).astype(jnp.bfloat16),
         jnp.broadcast_to(chi, (NUM_SEG, 4)),
         jnp.broadcast_to(clo, (NUM_SEG, 4))], axis=1)


def _accum(s_ref, st):
    g = pl.program_id(0)

    @pl.when(g == 0)
    def _():
        s_ref[...] = st

    @pl.when(g > 0)
    def _():
        s_ref[...] += st


def _normalize(hb, mt, t_ref):
    r = _dot_t(mt, t_ref[:, 0:192])
    rsd = r[:, 0:64] + r[:, 64:128]
    return hb.astype(jnp.float32) * rsd - r[:, 128:192]


def _first_kernel(x_ref, ir_ref, w_ref, b_ref, h_ref, t_ref, s_ref, *,
                  blk, nb):
    hb = _lrelu(_dot(x_ref[...], w_ref[...]) + b_ref[...]).astype(jnp.bfloat16)
    h_ref[...] = hb
    mt = _mask_t(ir_ref, blk)
    hh = jnp.concatenate(
        [hb, hb * hb, jnp.ones((blk, 8), jnp.bfloat16)], axis=1)
    _accum(s_ref, _dot(mt, hh))

    @pl.when(pl.program_id(0) == nb - 1)
    def _():
        cnt = s_ref[:, 128:129]
        _finalize_table(s_ref, cnt, t_ref)


def _mid_kernel(h_ref, ir_ref, t_ref, w_ref, b_ref, ho_ref, to_ref,
                s_ref, *, blk, nb):
    mt = _mask_t(ir_ref, blk)
    z = _normalize(h_ref[...], mt, t_ref)
    hb = _lrelu(_dot(z, w_ref[...]) + b_ref[...]).astype(jnp.bfloat16)
    ho_ref[...] = hb
    hh = jnp.concatenate([hb, hb * hb], axis=1)
    _accum(s_ref, _dot(mt, hh))

    @pl.when(pl.program_id(0) == nb - 1)
    def _():
        cnt = (t_ref[:, 192:193].astype(jnp.float32)
               + t_ref[:, 196:197].astype(jnp.float32))
        _finalize_table(s_ref, cnt, to_ref)


def _last_kernel(h_ref, ir_ref, t_ref, w_ref, b_ref, o_ref, *, blk):
    z = _normalize(h_ref[...], _mask_t(ir_ref, blk), t_ref)
    o_ref[...] = _dot(z, w_ref[...]) + b_ref[...]


def kernel(x, i, W1, b1, W2, b2, W3, b3, W4, b4):
    n, d = x.shape
    blk1 = _pick_block(n)
    blk2 = _pick_block_wide(n)
    nb1 = n // blk1
    nb2 = n // blk2
    i_row = i.reshape(1, n)

    row_spec = lambda b, w: pl.BlockSpec((b, w), lambda g: (g, 0))
    ir_spec = lambda b: pl.BlockSpec((1, b), lambda g: (0, g))
    full = lambda *s: pl.BlockSpec(s, lambda g: (0,) * len(s))
    tab_shape = jax.ShapeDtypeStruct((NUM_SEG, TAB_W), jnp.bfloat16)
    h_shape = jax.ShapeDtypeStruct((n, 64), jnp.bfloat16)
    stats_scratch = lambda w: [pltpu.VMEM((NUM_SEG, w), jnp.float32)]

    h1, t1 = pl.pallas_call(
        functools.partial(_first_kernel, blk=blk1, nb=nb1),
        grid=(nb1,),
        in_specs=[row_spec(blk1, d), ir_spec(blk1), full(d, 64), full(1, 64)],
        out_specs=[row_spec(blk1, 64), full(NUM_SEG, TAB_W)],
        out_shape=[h_shape, tab_shape],
        scratch_shapes=stats_scratch(136),
    )(x, i_row, W1.T, b1.reshape(1, 64))

    mid = pl.pallas_call(
        functools.partial(_mid_kernel, blk=blk2, nb=nb2),
        grid=(nb2,),
        in_specs=[row_spec(blk2, 64), ir_spec(blk2), full(NUM_SEG, TAB_W),
                  full(64, 64), full(1, 64)],
        out_specs=[row_spec(blk2, 64), full(NUM_SEG, TAB_W)],
        out_shape=[h_shape, tab_shape],
        scratch_shapes=stats_scratch(128),
    )
    h2, t2 = mid(h1, i_row, t1, W2.T, b2.reshape(1, 64))
    h3, t3 = mid(h2, i_row, t2, W3.T, b3.reshape(1, 64))

    out = pl.pallas_call(
        functools.partial(_last_kernel, blk=blk1),
        grid=(nb1,),
        in_specs=[row_spec(blk1, 64), ir_spec(blk1), full(NUM_SEG, TAB_W),
                  full(64, 3), full(1, 3)],
        out_specs=row_spec(blk1, 3),
        out_shape=jax.ShapeDtypeStruct((n, 3), jnp.float32),
    )(h3, i_row, t3, W4.T, b4.reshape(1, 3))
    return out
